# baseline (speedup 1.0000x reference)
import jax
import jax.numpy as jnp
from jax.experimental import pallas as pl
from jax.experimental.pallas import tpu as pltpu

N = 8192
G = 256
NPG = 32
BLK = 256
GPB = BLK // NPG
NB = N // BLK
EPS = 1e-5
H1 = 256
H2 = 256
P = 2 * (H1 + H2)


def _fused(x_ref, adj_ref, w1_ref, ws1_ref, b1_ref, g1_ref, bb1_ref,
           w2_ref, ws2_ref, b2_ref, g2_ref, bb2_ref,
           pooled_ref,
           r1_s, r2_s, adj_s, st1_s, st2_s):
    i = pl.program_id(0)

    def bn_affine(st_ref, g, b):
        m = st_ref[0:1, :] / N
        v = st_ref[1:2, :] / N - m * m
        scale = g * jax.lax.rsqrt(v + EPS)
        return scale, b - m * scale

    @pl.when(i < NB)
    def _phase_a():
        blk = i
        xb = x_ref[...]
        ab = adj_ref[...]
        adj_s[pl.ds(blk * BLK, BLK), :] = ab
        u = jnp.dot(xb, w1_ref[...], preferred_element_type=jnp.float32)
        o = (jnp.dot(ab, u, preferred_element_type=jnp.float32) + b1_ref[...]
             + jnp.dot(xb, ws1_ref[...], preferred_element_type=jnp.float32))
        r = jnp.maximum(o, 0.0)
        r1_s[pl.ds(blk * BLK, BLK), :] = r

        @pl.when(i == 0)
        def _():
            st1_s[...] = jnp.zeros_like(st1_s)

        st1_s[0:1, :] += jnp.sum(r, axis=0, keepdims=True)
        st1_s[1:2, :] += jnp.sum(r * r, axis=0, keepdims=True)

    @pl.when((i >= NB) & (i < 2 * NB))
    def _phase_b():
        blk = i - NB
        scale, shift = bn_affine(st1_s, g1_ref[...], bb1_ref[...])
        h = r1_s[pl.ds(blk * BLK, BLK), :] * scale + shift
        ab = adj_s[pl.ds(blk * BLK, BLK), :]
        u = jnp.dot(h, w2_ref[...], preferred_element_type=jnp.float32)
        o = (jnp.dot(ab, u, preferred_element_type=jnp.float32) + b2_ref[...]
             + jnp.dot(h, ws2_ref[...], preferred_element_type=jnp.float32))
        r = jnp.maximum(o, 0.0)
        r2_s[pl.ds(blk * BLK, BLK), :] = r

        @pl.when(i == NB)
        def _():
            st2_s[...] = jnp.zeros_like(st2_s)

        st2_s[0:1, :] += jnp.sum(r, axis=0, keepdims=True)
        st2_s[1:2, :] += jnp.sum(r * r, axis=0, keepdims=True)

    @pl.when(i >= 2 * NB)
    def _phase_c():
        blk = i - 2 * NB
        sc1, sh1 = bn_affine(st1_s, g1_ref[...], bb1_ref[...])
        sc2, sh2 = bn_affine(st2_s, g2_ref[...], bb2_ref[...])
        e1 = r1_s[pl.ds(blk * BLK, BLK), :] * sc1 + sh1
        h2 = r2_s[pl.ds(blk * BLK, BLK), :] * sc2 + sh2
        for g in range(GPB):
            e1g = e1[g * NPG:(g + 1) * NPG, :]
            h2g = h2[g * NPG:(g + 1) * NPG, :]
            pooled_ref[g:g + 1, 0:H1] = jnp.max(e1g, axis=0, keepdims=True)
            pooled_ref[g:g + 1, H1:H1 + H2] = jnp.max(h2g, axis=0, keepdims=True)
            pooled_ref[g:g + 1, H1 + H2:2 * H1 + H2] = (
                jnp.sum(e1g, axis=0, keepdims=True) / NPG)
            pooled_ref[g:g + 1, 2 * H1 + H2:P] = (
                jnp.sum(h2g, axis=0, keepdims=True) / NPG)


def _head(pooled_ref, g0_ref, b0_ref, w1_ref, b1_ref, g1_ref, bb1_ref,
          w2_ref, b2_ref, g2_ref, bb2_ref, w3_ref, b3_ref, wc_ref, bc_ref,
          out_ref, outc_ref, fp_ref):
    def bn(t, g, b):
        m = jnp.mean(t, axis=0, keepdims=True)
        v = jnp.mean(t * t, axis=0, keepdims=True) - m * m
        return (t - m) * (g * jax.lax.rsqrt(v + EPS)) + b

    p = bn(pooled_ref[...], g0_ref[...], b0_ref[...])
    p = jnp.maximum(jnp.dot(p, w1_ref[...], preferred_element_type=jnp.float32)
                    + b1_ref[...], 0.0)
    p = bn(p, g1_ref[...], bb1_ref[...])
    p = jnp.maximum(jnp.dot(p, w2_ref[...], preferred_element_type=jnp.float32)
                    + b2_ref[...], 0.0)
    fp = bn(p, g2_ref[...], bb2_ref[...])
    fp_ref[...] = fp
    out_ref[...] = jnp.dot(fp, w3_ref[...], preferred_element_type=jnp.float32) + b3_ref[...]
    outc_ref[...] = jnp.dot(fp, wc_ref[...], preferred_element_type=jnp.float32) + bc_ref[...]


def kernel(x, adj, slice_list, W1, Wskip1, b1, W2, Wskip2, b2, bng1_g, bng1_b,
           bng2_g, bng2_b, bn0_g, bn0_b, lin1_W, lin1_b, bn1_g, bn1_b, lin2_W,
           lin2_b, bn2_g, bn2_b, lin3_W, lin3_b, cat_W, cat_b):
    D = x.shape[1]
    L1 = lin2_W.shape[1]
    L2 = lin3_W.shape[1]
    NC = cat_W.shape[1]

    row = lambda a: a.reshape(1, -1)
    full = lambda a: pl.BlockSpec(a.shape, lambda i: (0,) * a.ndim)

    def x_map(i):
        j = jnp.minimum(i, NB - 1)
        return (j, 0)

    def adj_map(i):
        j = jnp.minimum(i, NB - 1)
        return (j, j)

    def pooled_map(i):
        j = jnp.maximum(i - 2 * NB, 0)
        return (j, 0)

    args = (x, adj, W1, Wskip1, row(b1), row(bng1_g), row(bng1_b),
            W2, Wskip2, row(b2), row(bng2_g), row(bng2_b))

    in_specs = [
        pl.BlockSpec((BLK, D), x_map),
        pl.BlockSpec((BLK, BLK), adj_map),
    ] + [full(a) for a in args[2:]]

    pooled = pl.pallas_call(
        _fused,
        grid=(3 * NB,),
        in_specs=in_specs,
        out_specs=pl.BlockSpec((GPB, P), pooled_map),
        out_shape=jax.ShapeDtypeStruct((G, P), jnp.float32),
        scratch_shapes=[
            pltpu.VMEM((N, H1), jnp.float32),
            pltpu.VMEM((N, H2), jnp.float32),
            pltpu.VMEM((N, BLK), jnp.float32),
            pltpu.VMEM((8, H1), jnp.float32),
            pltpu.VMEM((8, H2), jnp.float32),
        ],
        compiler_params=pltpu.CompilerParams(
            dimension_semantics=("arbitrary",),
        ),
    )(*args)

    hargs = (pooled, row(bn0_g), row(bn0_b), lin1_W, row(lin1_b), row(bn1_g),
             row(bn1_b), lin2_W, row(lin2_b), row(bn2_g), row(bn2_b), lin3_W,
             row(lin3_b), cat_W, row(cat_b))
    full0 = lambda a: pl.BlockSpec(a.shape, lambda: (0,) * a.ndim)
    out, out_class, fp = pl.pallas_call(
        _head,
        in_specs=[full0(a) for a in hargs],
        out_specs=[
            pl.BlockSpec((G, L2), lambda: (0, 0)),
            pl.BlockSpec((G, NC), lambda: (0, 0)),
            pl.BlockSpec((G, L1), lambda: (0, 0)),
        ],
        out_shape=[
            jax.ShapeDtypeStruct((G, L2), jnp.float32),
            jax.ShapeDtypeStruct((G, NC), jnp.float32),
            jax.ShapeDtypeStruct((G, L1), jnp.float32),
        ],
    )(*hargs)

    return (out, out_class, fp)


# speedup vs baseline: 1.0439x; 1.0439x over previous
import jax
import jax.numpy as jnp
from jax.experimental import pallas as pl
from jax.experimental.pallas import tpu as pltpu

N = 8192
G = 256
NPG = 32
BLK = 256
GPB = BLK // NPG
NB = N // BLK
EPS = 1e-5
H1 = 256
H2 = 256
P = 2 * (H1 + H2)
BF = jnp.bfloat16


def _dot(a, b):
    return jnp.dot(a, b, preferred_element_type=jnp.float32)


def _fused(x_ref, adj_ref, w1_ref, ws1_ref, b1_ref, g1_ref, bb1_ref,
           w2_ref, ws2_ref, b2_ref, g2_ref, bb2_ref,
           g0_ref, b0_ref, l1w_ref, l1b_ref, hg1_ref, hb1_ref,
           l2w_ref, l2b_ref, hg2_ref, hb2_ref, l3w_ref, l3b_ref,
           cw_ref, cb_ref,
           out_ref, outc_ref, fp_ref,
           r1_s, r2_s, adj_s, pooled_s, st1_s, st2_s):
    i = pl.program_id(0)

    def bn_affine(st_ref, g, b):
        m = st_ref[0:1, :] / N
        v = st_ref[1:2, :] / N - m * m
        scale = g * jax.lax.rsqrt(v + EPS)
        return scale, b - m * scale

    @pl.when(i < NB)
    def _phase_a():
        blk = i
        xb = x_ref[...].astype(BF)
        ab = adj_ref[...].astype(BF)
        adj_s[pl.ds(blk * BLK, BLK), :] = ab
        u = _dot(xb, w1_ref[...].astype(BF)).astype(BF)
        o = (_dot(ab, u) + b1_ref[...] + _dot(xb, ws1_ref[...].astype(BF)))
        r = jnp.maximum(o, 0.0)
        r1_s[pl.ds(blk * BLK, BLK), :] = r

        @pl.when(i == 0)
        def _():
            st1_s[...] = jnp.zeros_like(st1_s)

        st1_s[0:1, :] += jnp.sum(r, axis=0, keepdims=True)
        st1_s[1:2, :] += jnp.sum(r * r, axis=0, keepdims=True)

    @pl.when((i >= NB) & (i < 2 * NB))
    def _phase_b():
        blk = i - NB
        scale, shift = bn_affine(st1_s, g1_ref[...], bb1_ref[...])
        h = (r1_s[pl.ds(blk * BLK, BLK), :] * scale + shift).astype(BF)
        ab = adj_s[pl.ds(blk * BLK, BLK), :]
        u = _dot(h, w2_ref[...].astype(BF)).astype(BF)
        o = (_dot(ab, u) + b2_ref[...] + _dot(h, ws2_ref[...].astype(BF)))
        r = jnp.maximum(o, 0.0)
        r2_s[pl.ds(blk * BLK, BLK), :] = r

        @pl.when(i == NB)
        def _():
            st2_s[...] = jnp.zeros_like(st2_s)

        st2_s[0:1, :] += jnp.sum(r, axis=0, keepdims=True)
        st2_s[1:2, :] += jnp.sum(r * r, axis=0, keepdims=True)

    @pl.when((i >= 2 * NB) & (i < 3 * NB))
    def _phase_c():
        blk = i - 2 * NB
        sc1, sh1 = bn_affine(st1_s, g1_ref[...], bb1_ref[...])
        sc2, sh2 = bn_affine(st2_s, g2_ref[...], bb2_ref[...])
        e1 = r1_s[pl.ds(blk * BLK, BLK), :] * sc1 + sh1
        h2 = r2_s[pl.ds(blk * BLK, BLK), :] * sc2 + sh2
        for g in range(GPB):
            e1g = e1[g * NPG:(g + 1) * NPG, :]
            h2g = h2[g * NPG:(g + 1) * NPG, :]
            row = blk * GPB + g
            pooled_s[pl.ds(row, 1), 0:H1] = jnp.max(e1g, axis=0, keepdims=True)
            pooled_s[pl.ds(row, 1), H1:H1 + H2] = jnp.max(h2g, axis=0, keepdims=True)
            pooled_s[pl.ds(row, 1), H1 + H2:2 * H1 + H2] = (
                jnp.sum(e1g, axis=0, keepdims=True) / NPG)
            pooled_s[pl.ds(row, 1), 2 * H1 + H2:P] = (
                jnp.sum(h2g, axis=0, keepdims=True) / NPG)

    @pl.when(i == 3 * NB)
    def _phase_d():
        def bn(t, g, b):
            m = jnp.mean(t, axis=0, keepdims=True)
            v = jnp.mean(t * t, axis=0, keepdims=True) - m * m
            return (t - m) * (g * jax.lax.rsqrt(v + EPS)) + b

        p = bn(pooled_s[...], g0_ref[...], b0_ref[...])
        p = jnp.maximum(_dot(p.astype(BF), l1w_ref[...].astype(BF))
                        + l1b_ref[...], 0.0)
        p = bn(p, hg1_ref[...], hb1_ref[...])
        p = jnp.maximum(_dot(p.astype(BF), l2w_ref[...].astype(BF))
                        + l2b_ref[...], 0.0)
        fp = bn(p, hg2_ref[...], hb2_ref[...])
        fp_ref[...] = fp
        fpb = fp.astype(BF)
        out_ref[...] = _dot(fpb, l3w_ref[...].astype(BF)) + l3b_ref[...]
        outc_ref[...] = _dot(fpb, cw_ref[...].astype(BF)) + cb_ref[...]


def kernel(x, adj, slice_list, W1, Wskip1, b1, W2, Wskip2, b2, bng1_g, bng1_b,
           bng2_g, bng2_b, bn0_g, bn0_b, lin1_W, lin1_b, bn1_g, bn1_b, lin2_W,
           lin2_b, bn2_g, bn2_b, lin3_W, lin3_b, cat_W, cat_b):
    D = x.shape[1]
    L1 = lin2_W.shape[1]
    L2 = lin3_W.shape[1]
    NC = cat_W.shape[1]

    row = lambda a: a.reshape(1, -1)
    full = lambda a: pl.BlockSpec(a.shape, lambda i: (0,) * a.ndim)

    def x_map(i):
        j = jnp.minimum(i, NB - 1)
        return (j, 0)

    def adj_map(i):
        j = jnp.minimum(i, NB - 1)
        return (j, j)

    args = (x, adj, W1, Wskip1, row(b1), row(bng1_g), row(bng1_b),
            W2, Wskip2, row(b2), row(bng2_g), row(bng2_b),
            row(bn0_g), row(bn0_b), lin1_W, row(lin1_b), row(bn1_g), row(bn1_b),
            lin2_W, row(lin2_b), row(bn2_g), row(bn2_b), lin3_W, row(lin3_b),
            cat_W, row(cat_b))

    in_specs = [
        pl.BlockSpec((BLK, D), x_map),
        pl.BlockSpec((BLK, BLK), adj_map),
    ] + [full(a) for a in args[2:]]

    out, out_class, fp = pl.pallas_call(
        _fused,
        grid=(3 * NB + 1,),
        in_specs=in_specs,
        out_specs=[
            pl.BlockSpec((G, L2), lambda i: (0, 0)),
            pl.BlockSpec((G, NC), lambda i: (0, 0)),
            pl.BlockSpec((G, L1), lambda i: (0, 0)),
        ],
        out_shape=[
            jax.ShapeDtypeStruct((G, L2), jnp.float32),
            jax.ShapeDtypeStruct((G, NC), jnp.float32),
            jax.ShapeDtypeStruct((G, L1), jnp.float32),
        ],
        scratch_shapes=[
            pltpu.VMEM((N, H1), jnp.float32),
            pltpu.VMEM((N, H2), jnp.float32),
            pltpu.VMEM((N, BLK), BF),
            pltpu.VMEM((G, P), jnp.float32),
            pltpu.VMEM((8, H1), jnp.float32),
            pltpu.VMEM((8, H2), jnp.float32),
        ],
        compiler_params=pltpu.CompilerParams(
            dimension_semantics=("arbitrary",),
        ),
    )(*args)

    return (out, out_class, fp)


# speedup vs baseline: 1.3735x; 1.3157x over previous
import jax
import jax.numpy as jnp
from jax.experimental import pallas as pl
from jax.experimental.pallas import tpu as pltpu

N = 8192
G = 256
NPG = 32
BLK = 512
GPB = BLK // NPG
NB = N // BLK
EPS = 1e-5
H1 = 256
H2 = 256
P = 2 * (H1 + H2)


def _fused(x_ref, adj_ref, w1_ref, ws1_ref, b1_ref, g1_ref, bb1_ref,
           w2_ref, ws2_ref, b2_ref, g2_ref, bb2_ref,
           pooled_ref,
           r1_s, r2_s, adj_s, st1_s, st2_s):
    i = pl.program_id(0)

    def bn_affine(st_ref, g, b):
        m = st_ref[0:1, :] / N
        v = st_ref[1:2, :] / N - m * m
        scale = g * jax.lax.rsqrt(v + EPS)
        return scale, b - m * scale

    @pl.when(i < NB)
    def _phase_a():
        blk = i
        xb = x_ref[...]
        ab = adj_ref[...]
        adj_s[pl.ds(blk * BLK, BLK), :] = ab
        u = jnp.dot(xb, w1_ref[...], preferred_element_type=jnp.float32)
        o = (jnp.dot(ab, u, preferred_element_type=jnp.float32) + b1_ref[...]
             + jnp.dot(xb, ws1_ref[...], preferred_element_type=jnp.float32))
        r = jnp.maximum(o, 0.0)
        r1_s[pl.ds(blk * BLK, BLK), :] = r

        @pl.when(i == 0)
        def _():
            st1_s[...] = jnp.zeros_like(st1_s)

        st1_s[0:1, :] += jnp.sum(r, axis=0, keepdims=True)
        st1_s[1:2, :] += jnp.sum(r * r, axis=0, keepdims=True)

    @pl.when((i >= NB) & (i < 2 * NB))
    def _phase_b():
        blk = i - NB
        scale, shift = bn_affine(st1_s, g1_ref[...], bb1_ref[...])
        h = r1_s[pl.ds(blk * BLK, BLK), :] * scale + shift
        ab = adj_s[pl.ds(blk * BLK, BLK), :]
        u = jnp.dot(h, w2_ref[...], preferred_element_type=jnp.float32)
        o = (jnp.dot(ab, u, preferred_element_type=jnp.float32) + b2_ref[...]
             + jnp.dot(h, ws2_ref[...], preferred_element_type=jnp.float32))
        r = jnp.maximum(o, 0.0)
        r2_s[pl.ds(blk * BLK, BLK), :] = r

        @pl.when(i == NB)
        def _():
            st2_s[...] = jnp.zeros_like(st2_s)

        st2_s[0:1, :] += jnp.sum(r, axis=0, keepdims=True)
        st2_s[1:2, :] += jnp.sum(r * r, axis=0, keepdims=True)

    @pl.when(i >= 2 * NB)
    def _phase_c():
        blk = i - 2 * NB
        sc1, sh1 = bn_affine(st1_s, g1_ref[...], bb1_ref[...])
        sc2, sh2 = bn_affine(st2_s, g2_ref[...], bb2_ref[...])
        e1 = r1_s[pl.ds(blk * BLK, BLK), :] * sc1 + sh1
        h2 = r2_s[pl.ds(blk * BLK, BLK), :] * sc2 + sh2
        for g in range(GPB):
            e1g = e1[g * NPG:(g + 1) * NPG, :]
            h2g = h2[g * NPG:(g + 1) * NPG, :]
            pooled_ref[g:g + 1, 0:H1] = jnp.max(e1g, axis=0, keepdims=True)
            pooled_ref[g:g + 1, H1:H1 + H2] = jnp.max(h2g, axis=0, keepdims=True)
            pooled_ref[g:g + 1, H1 + H2:2 * H1 + H2] = (
                jnp.sum(e1g, axis=0, keepdims=True) / NPG)
            pooled_ref[g:g + 1, 2 * H1 + H2:P] = (
                jnp.sum(h2g, axis=0, keepdims=True) / NPG)


def _head(pooled_ref, g0_ref, b0_ref, w1_ref, b1_ref, g1_ref, bb1_ref,
          w2_ref, b2_ref, g2_ref, bb2_ref, w3_ref, b3_ref, wc_ref, bc_ref,
          out_ref, outc_ref, fp_ref):
    def bn(t, g, b):
        m = jnp.mean(t, axis=0, keepdims=True)
        v = jnp.mean(t * t, axis=0, keepdims=True) - m * m
        return (t - m) * (g * jax.lax.rsqrt(v + EPS)) + b

    p = bn(pooled_ref[...], g0_ref[...], b0_ref[...])
    p = jnp.maximum(jnp.dot(p, w1_ref[...], preferred_element_type=jnp.float32)
                    + b1_ref[...], 0.0)
    p = bn(p, g1_ref[...], bb1_ref[...])
    p = jnp.maximum(jnp.dot(p, w2_ref[...], preferred_element_type=jnp.float32)
                    + b2_ref[...], 0.0)
    fp = bn(p, g2_ref[...], bb2_ref[...])
    fp_ref[...] = fp
    out_ref[...] = jnp.dot(fp, w3_ref[...], preferred_element_type=jnp.float32) + b3_ref[...]
    outc_ref[...] = jnp.dot(fp, wc_ref[...], preferred_element_type=jnp.float32) + bc_ref[...]


def kernel(x, adj, slice_list, W1, Wskip1, b1, W2, Wskip2, b2, bng1_g, bng1_b,
           bng2_g, bng2_b, bn0_g, bn0_b, lin1_W, lin1_b, bn1_g, bn1_b, lin2_W,
           lin2_b, bn2_g, bn2_b, lin3_W, lin3_b, cat_W, cat_b):
    D = x.shape[1]
    L1 = lin2_W.shape[1]
    L2 = lin3_W.shape[1]
    NC = cat_W.shape[1]

    row = lambda a: a.reshape(1, -1)
    full = lambda a: pl.BlockSpec(a.shape, lambda i: (0,) * a.ndim)

    def x_map(i):
        j = jnp.minimum(i, NB - 1)
        return (j, 0)

    def adj_map(i):
        j = jnp.minimum(i, NB - 1)
        return (j, j)

    def pooled_map(i):
        j = jnp.maximum(i - 2 * NB, 0)
        return (j, 0)

    args = (x, adj, W1, Wskip1, row(b1), row(bng1_g), row(bng1_b),
            W2, Wskip2, row(b2), row(bng2_g), row(bng2_b))

    in_specs = [
        pl.BlockSpec((BLK, D), x_map),
        pl.BlockSpec((BLK, BLK), adj_map),
    ] + [full(a) for a in args[2:]]

    pooled = pl.pallas_call(
        _fused,
        grid=(3 * NB,),
        in_specs=in_specs,
        out_specs=pl.BlockSpec((GPB, P), pooled_map),
        out_shape=jax.ShapeDtypeStruct((G, P), jnp.float32),
        scratch_shapes=[
            pltpu.VMEM((N, H1), jnp.float32),
            pltpu.VMEM((N, H2), jnp.float32),
            pltpu.VMEM((N, BLK), jnp.float32),
            pltpu.VMEM((8, H1), jnp.float32),
            pltpu.VMEM((8, H2), jnp.float32),
        ],
        compiler_params=pltpu.CompilerParams(
            dimension_semantics=("arbitrary",),
        ),
    )(*args)

    hargs = (pooled, row(bn0_g), row(bn0_b), lin1_W, row(lin1_b), row(bn1_g),
             row(bn1_b), lin2_W, row(lin2_b), row(bn2_g), row(bn2_b), lin3_W,
             row(lin3_b), cat_W, row(cat_b))
    full0 = lambda a: pl.BlockSpec(a.shape, lambda: (0,) * a.ndim)
    out, out_class, fp = pl.pallas_call(
        _head,
        in_specs=[full0(a) for a in hargs],
        out_specs=[
            pl.BlockSpec((G, L2), lambda: (0, 0)),
            pl.BlockSpec((G, NC), lambda: (0, 0)),
            pl.BlockSpec((G, L1), lambda: (0, 0)),
        ],
        out_shape=[
            jax.ShapeDtypeStruct((G, L2), jnp.float32),
            jax.ShapeDtypeStruct((G, NC), jnp.float32),
            jax.ShapeDtypeStruct((G, L1), jnp.float32),
        ],
    )(*hargs)

    return (out, out_class, fp)


# speedup vs baseline: 1.5500x; 1.1285x over previous
import jax
import jax.numpy as jnp
from jax.experimental import pallas as pl
from jax.experimental.pallas import tpu as pltpu

N = 8192
G = 256
NPG = 32
BLK = 1024
GPB = BLK // NPG
NB = N // BLK
EPS = 1e-5
H1 = 256
H2 = 256
P = 2 * (H1 + H2)


def _fused(x_ref, adj_ref, w1_ref, ws1_ref, b1_ref, g1_ref, bb1_ref,
           w2_ref, ws2_ref, b2_ref, g2_ref, bb2_ref,
           pooled_ref,
           r1_s, r2_s, adj_s, st1_s, st2_s):
    i = pl.program_id(0)

    def bn_affine(st_ref, g, b):
        m = st_ref[0:1, :] / N
        v = st_ref[1:2, :] / N - m * m
        scale = g * jax.lax.rsqrt(v + EPS)
        return scale, b - m * scale

    @pl.when(i < NB)
    def _phase_a():
        blk = i
        xb = x_ref[...]
        ab = adj_ref[...]
        adj_s[pl.ds(blk * BLK, BLK), :] = ab.astype(jnp.bfloat16)
        u = jnp.dot(xb, w1_ref[...], preferred_element_type=jnp.float32)
        o = (jnp.dot(ab, u, preferred_element_type=jnp.float32) + b1_ref[...]
             + jnp.dot(xb, ws1_ref[...], preferred_element_type=jnp.float32))
        r = jnp.maximum(o, 0.0)
        r1_s[pl.ds(blk * BLK, BLK), :] = r

        @pl.when(i == 0)
        def _():
            st1_s[...] = jnp.zeros_like(st1_s)

        st1_s[0:1, :] += jnp.sum(r, axis=0, keepdims=True)
        st1_s[1:2, :] += jnp.sum(r * r, axis=0, keepdims=True)

    @pl.when((i >= NB) & (i < 2 * NB))
    def _phase_b():
        blk = i - NB
        scale, shift = bn_affine(st1_s, g1_ref[...], bb1_ref[...])
        h = r1_s[pl.ds(blk * BLK, BLK), :] * scale + shift
        ab = adj_s[pl.ds(blk * BLK, BLK), :].astype(jnp.float32)
        u = jnp.dot(h, w2_ref[...], preferred_element_type=jnp.float32)
        o = (jnp.dot(ab, u, preferred_element_type=jnp.float32) + b2_ref[...]
             + jnp.dot(h, ws2_ref[...], preferred_element_type=jnp.float32))
        r = jnp.maximum(o, 0.0)
        r2_s[pl.ds(blk * BLK, BLK), :] = r

        @pl.when(i == NB)
        def _():
            st2_s[...] = jnp.zeros_like(st2_s)

        st2_s[0:1, :] += jnp.sum(r, axis=0, keepdims=True)
        st2_s[1:2, :] += jnp.sum(r * r, axis=0, keepdims=True)

    @pl.when(i >= 2 * NB)
    def _phase_c():
        blk = i - 2 * NB
        sc1, sh1 = bn_affine(st1_s, g1_ref[...], bb1_ref[...])
        sc2, sh2 = bn_affine(st2_s, g2_ref[...], bb2_ref[...])
        e1 = r1_s[pl.ds(blk * BLK, BLK), :] * sc1 + sh1
        h2 = r2_s[pl.ds(blk * BLK, BLK), :] * sc2 + sh2
        for g in range(GPB):
            e1g = e1[g * NPG:(g + 1) * NPG, :]
            h2g = h2[g * NPG:(g + 1) * NPG, :]
            pooled_ref[g:g + 1, 0:H1] = jnp.max(e1g, axis=0, keepdims=True)
            pooled_ref[g:g + 1, H1:H1 + H2] = jnp.max(h2g, axis=0, keepdims=True)
            pooled_ref[g:g + 1, H1 + H2:2 * H1 + H2] = (
                jnp.sum(e1g, axis=0, keepdims=True) / NPG)
            pooled_ref[g:g + 1, 2 * H1 + H2:P] = (
                jnp.sum(h2g, axis=0, keepdims=True) / NPG)


def _head(pooled_ref, g0_ref, b0_ref, w1_ref, b1_ref, g1_ref, bb1_ref,
          w2_ref, b2_ref, g2_ref, bb2_ref, w3_ref, b3_ref, wc_ref, bc_ref,
          out_ref, outc_ref, fp_ref):
    def bn(t, g, b):
        m = jnp.mean(t, axis=0, keepdims=True)
        v = jnp.mean(t * t, axis=0, keepdims=True) - m * m
        return (t - m) * (g * jax.lax.rsqrt(v + EPS)) + b

    p = bn(pooled_ref[...], g0_ref[...], b0_ref[...])
    p = jnp.maximum(jnp.dot(p, w1_ref[...], preferred_element_type=jnp.float32)
                    + b1_ref[...], 0.0)
    p = bn(p, g1_ref[...], bb1_ref[...])
    p = jnp.maximum(jnp.dot(p, w2_ref[...], preferred_element_type=jnp.float32)
                    + b2_ref[...], 0.0)
    fp = bn(p, g2_ref[...], bb2_ref[...])
    fp_ref[...] = fp
    out_ref[...] = jnp.dot(fp, w3_ref[...], preferred_element_type=jnp.float32) + b3_ref[...]
    outc_ref[...] = jnp.dot(fp, wc_ref[...], preferred_element_type=jnp.float32) + bc_ref[...]


def kernel(x, adj, slice_list, W1, Wskip1, b1, W2, Wskip2, b2, bng1_g, bng1_b,
           bng2_g, bng2_b, bn0_g, bn0_b, lin1_W, lin1_b, bn1_g, bn1_b, lin2_W,
           lin2_b, bn2_g, bn2_b, lin3_W, lin3_b, cat_W, cat_b):
    D = x.shape[1]
    L1 = lin2_W.shape[1]
    L2 = lin3_W.shape[1]
    NC = cat_W.shape[1]

    row = lambda a: a.reshape(1, -1)
    full = lambda a: pl.BlockSpec(a.shape, lambda i: (0,) * a.ndim)

    def x_map(i):
        j = jnp.minimum(i, NB - 1)
        return (j, 0)

    def adj_map(i):
        j = jnp.minimum(i, NB - 1)
        return (j, j)

    def pooled_map(i):
        j = jnp.maximum(i - 2 * NB, 0)
        return (j, 0)

    args = (x, adj, W1, Wskip1, row(b1), row(bng1_g), row(bng1_b),
            W2, Wskip2, row(b2), row(bng2_g), row(bng2_b))

    in_specs = [
        pl.BlockSpec((BLK, D), x_map),
        pl.BlockSpec((BLK, BLK), adj_map),
    ] + [full(a) for a in args[2:]]

    pooled = pl.pallas_call(
        _fused,
        grid=(3 * NB,),
        in_specs=in_specs,
        out_specs=pl.BlockSpec((GPB, P), pooled_map),
        out_shape=jax.ShapeDtypeStruct((G, P), jnp.float32),
        scratch_shapes=[
            pltpu.VMEM((N, H1), jnp.float32),
            pltpu.VMEM((N, H2), jnp.float32),
            pltpu.VMEM((N, BLK), jnp.bfloat16),
            pltpu.VMEM((8, H1), jnp.float32),
            pltpu.VMEM((8, H2), jnp.float32),
        ],
        compiler_params=pltpu.CompilerParams(
            dimension_semantics=("arbitrary",),
        ),
    )(*args)

    hargs = (pooled, row(bn0_g), row(bn0_b), lin1_W, row(lin1_b), row(bn1_g),
             row(bn1_b), lin2_W, row(lin2_b), row(bn2_g), row(bn2_b), lin3_W,
             row(lin3_b), cat_W, row(cat_b))
    full0 = lambda a: pl.BlockSpec(a.shape, lambda: (0,) * a.ndim)
    out, out_class, fp = pl.pallas_call(
        _head,
        in_specs=[full0(a) for a in hargs],
        out_specs=[
            pl.BlockSpec((G, L2), lambda: (0, 0)),
            pl.BlockSpec((G, NC), lambda: (0, 0)),
            pl.BlockSpec((G, L1), lambda: (0, 0)),
        ],
        out_shape=[
            jax.ShapeDtypeStruct((G, L2), jnp.float32),
            jax.ShapeDtypeStruct((G, NC), jnp.float32),
            jax.ShapeDtypeStruct((G, L1), jnp.float32),
        ],
    )(*hargs)

    return (out, out_class, fp)
